# baseline (speedup 1.0000x reference)
import functools

import jax
import jax.numpy as jnp
from jax import lax
from jax.experimental import pallas as pl
from jax.experimental.pallas import tpu as pltpu
from jax.experimental.pallas import tpu_sc as plsc


def kernel(indices, table):
    B = indices.shape[0]
    V, D = table.shape
    info = plsc.get_sparse_core_info()
    NC = info.num_cores
    rows_per = B // NC
    CH = 512
    n_ch = rows_per // CH

    mesh = plsc.ScalarSubcoreMesh(axis_name="c", num_cores=NC)

    @functools.partial(
        pl.kernel,
        mesh=mesh,
        out_type=jax.ShapeDtypeStruct((B, D), jnp.float32),
        scratch_types=[
            pltpu.SMEM((CH,), jnp.int32),
            pltpu.VMEM_SHARED((rows_per, D), jnp.float32),
            pltpu.SemaphoreType.DMA,
            pltpu.SemaphoreType.DMA,
            pltpu.SemaphoreType.DMA,
        ],
    )
    def gather_kernel(idx_hbm, table_hbm, out_hbm, idx_s, out_sh, sem, sem2,
                      sem3):
        wid = lax.axis_index("c")
        base = wid * rows_per

        def chunk_body(c):
            pltpu.async_copy(idx_hbm.at[pl.ds(base + c * CH, CH)], idx_s,
                             sem2).wait()
            copies = []
            for j in range(CH):
                r = idx_s[j]
                copies.append(
                    pltpu.async_copy(table_hbm.at[r],
                                     out_sh.at[c * CH + j], sem))
            for cp in copies:
                cp.wait()

        pl.loop(0, n_ch)(chunk_body)
        pltpu.async_copy(out_sh, out_hbm.at[pl.ds(base, rows_per)],
                         sem3).wait()

    return gather_kernel(indices, table)


# speedup vs baseline: 1.0829x; 1.0829x over previous
import functools

import jax
import jax.numpy as jnp
from jax import lax
from jax.experimental import pallas as pl
from jax.experimental.pallas import tpu as pltpu
from jax.experimental.pallas import tpu_sc as plsc


def kernel(indices, table):
    B = indices.shape[0]
    V, D = table.shape
    info = plsc.get_sparse_core_info()
    NC = info.num_cores
    rows_per = B // NC
    CH = 512
    n_ch = rows_per // CH

    mesh = plsc.ScalarSubcoreMesh(axis_name="c", num_cores=NC)

    @functools.partial(
        pl.kernel,
        mesh=mesh,
        out_type=jax.ShapeDtypeStruct((B, D), jnp.float32),
        scratch_types=[
            pltpu.SMEM((CH,), jnp.int32),
            pltpu.SMEM((CH,), jnp.int32),
            pltpu.VMEM_SHARED((rows_per, D), jnp.float32),
            pltpu.SemaphoreType.DMA,
            pltpu.SemaphoreType.DMA,
            pltpu.SemaphoreType.DMA,
        ],
    )
    def gather_kernel(idx_hbm, table_hbm, out_hbm, idx_a, idx_b, out_sh, sem,
                      sem_idx, sem_out):
        wid = lax.axis_index("c")
        base = wid * rows_per

        pltpu.async_copy(idx_hbm.at[pl.ds(base, CH)], idx_a, sem_idx)

        def half(c, buf, nxt):
            pltpu.make_async_copy(idx_hbm.at[pl.ds(base, CH)], buf,
                                  sem_idx).wait()

            @pl.when(c + 1 < n_ch)
            def _():
                pltpu.async_copy(idx_hbm.at[pl.ds(base + (c + 1) * CH, CH)],
                                 nxt, sem_idx)

            for j in range(CH):
                r = buf[j]
                pltpu.make_async_copy(table_hbm.at[r],
                                      out_sh.at[c * CH + j], sem).start()
            pltpu.make_async_copy(table_hbm.at[pl.ds(0, CH)],
                                  out_sh.at[pl.ds(c * CH, CH)], sem).wait()

        def pair_body(p):
            half(2 * p, idx_a, idx_b)
            half(2 * p + 1, idx_b, idx_a)

        pl.loop(0, n_ch // 2)(pair_body)
        pltpu.async_copy(out_sh, out_hbm.at[pl.ds(base, rows_per)],
                         sem_out).wait()

    return gather_kernel(indices, table)
